# baseline (speedup 1.0000x reference)
import functools

import jax
import jax.numpy as jnp
from jax import lax
from jax.experimental import pallas as pl
from jax.experimental.pallas import tpu as pltpu
from jax.experimental.pallas import tpu_sc as plsc

D = 128
GROUP = 128
NC = 2
NS = 16
NW = NC * NS


def _load_worker_idx(src_hbm, dst_v, bw, q, r, wid):
    pltpu.sync_copy(src_hbm.at[pl.ds(bw * GROUP, q * GROUP)],
                    dst_v.at[pl.ds(0, q * GROUP)])
    if r:
        @pl.when(wid < r)
        def _():
            pltpu.sync_copy(src_hbm.at[pl.ds((bw + q) * GROUP, GROUP)],
                            dst_v.at[pl.ds(q * GROUP, GROUP)])


def _sc_hist(edge_hbm, zeros_hbm, out0_hbm, out1_hbm, cidx_v, cbuf_v, ones_v,
             acc, sem, *, q, r, per_sub):
    c = lax.axis_index("c")
    s = lax.axis_index("s")
    wid = c * NS + s
    bw = wid * q + jnp.minimum(wid, r)
    tw = q + (wid < r).astype(jnp.int32)
    for j in range(GROUP // 16):
        ones_v[pl.ds(j * 16, 16)] = jnp.ones((16,), jnp.float32)
    _load_worker_idx(edge_hbm.at[1], cidx_v, bw, q, r, wid)
    pltpu.sync_copy(zeros_hbm, acc.at[pl.ds(s * per_sub, per_sub)])
    plsc.subcore_barrier()

    def body(g, carry):
        for j in range(GROUP // 16):
            cbuf_v[pl.ds(j * 16, 16)] = cidx_v[pl.ds(g * GROUP + j * 16, 16)]
        pltpu.sync_copy(ones_v, acc.at[cbuf_v], add=True)
        return carry

    lax.fori_loop(0, tw, body, 0)
    plsc.subcore_barrier()

    @pl.when(c == 0)
    def _():
        pltpu.sync_copy(acc.at[pl.ds(s * per_sub, per_sub)],
                        out0_hbm.at[pl.ds(s * per_sub, per_sub)])

    @pl.when(c == 1)
    def _():
        pltpu.sync_copy(acc.at[pl.ds(s * per_sub, per_sub)],
                        out1_hbm.at[pl.ds(s * per_sub, per_sub)])


def _sc_pass(u_hbm, edge_hbm, zeros_hbm, out_hbm,
             cidx_v, ridx0, ridx1, rows0, rows1,
             gsem0, gsem1, rsem0, rsem1, ssem0, ssem1, acc,
             *, q, r, per_sub):
    c = lax.axis_index("c")
    s = lax.axis_index("s")
    wid = c * NS + s
    bw = wid * q + jnp.minimum(wid, r)
    tw = q + (wid < r).astype(jnp.int32)
    _load_worker_idx(edge_hbm.at[1], cidx_v, bw, q, r, wid)
    pltpu.sync_copy(zeros_hbm, acc.at[pl.ds(s * per_sub, per_sub)])
    plsc.subcore_barrier()

    bufs = ((rows0, gsem0, ridx0, rsem0, ssem0),
            (rows1, gsem1, ridx1, rsem1, ssem1))

    def gather_of(g, rows, gsem):
        return pltpu.make_async_copy(
            u_hbm.at[cidx_v.at[pl.ds(g * GROUP, GROUP)]], rows, gsem)

    def ridx_of(g, ridx, rsem):
        return pltpu.make_async_copy(
            edge_hbm.at[0, pl.ds((bw + g) * GROUP, GROUP)], ridx, rsem)

    def scatter_of(rows, ridx, ssem):
        return pltpu.make_async_copy(rows, acc.at[ridx], ssem)

    @pl.when(tw > 0)
    def _():
        ridx_of(0, ridx0, rsem0).start()
        gather_of(0, rows0, gsem0).start()

    def body(g, carry):
        for b in range(2):
            @pl.when(lax.rem(g, 2) == b)
            def _(b=b):
                rows, gsem, ridx, rsem, ssem = bufs[b]
                orows, ogsem, oridx, orsem, ossem = bufs[1 - b]

                gather_of(g, rows, gsem).wait()

                @pl.when(g >= 1)
                def _():
                    scatter_of(orows, oridx, ossem).wait()

                @pl.when(g + 1 < tw)
                def _():
                    ridx_of(g + 1, oridx, orsem).start()
                    gather_of(g + 1, orows, ogsem).start()

                ridx_of(g, ridx, rsem).wait()
                pltpu.async_copy(rows, acc.at[ridx], ssem, add=True)
        return carry

    lax.fori_loop(0, tw, body, 0)
    for b in range(2):
        @pl.when(jnp.logical_and(tw > 0, lax.rem(tw - 1, 2) == b))
        def _(b=b):
            rows, _gs, ridx, _rs, ssem = bufs[b]
            scatter_of(rows, ridx, ssem).wait()
    plsc.subcore_barrier()
    pltpu.sync_copy(acc.at[pl.ds(s * per_sub, per_sub)],
                    out_hbm.at[c, pl.ds(s * per_sub, per_sub)])


def _tc_prep(dis_ref, x_ref, u0_ref):
    n = x_ref.shape[0]
    n_pad = u0_ref.shape[0]
    u0_ref[pl.ds(0, n), :] = dis_ref[pl.ds(0, n), :] * x_ref[...]
    u0_ref[pl.ds(n, n_pad - n), :] = jnp.zeros((n_pad - n, D), jnp.float32)


def _tc_mid(part_ref, u0_ref, dis_ref, fo_ref, u1_ref):
    v1 = part_ref[0] + part_ref[1] + u0_ref[...]
    dis = dis_ref[...]
    fo = dis * v1
    fo_ref[...] = fo
    u1_ref[...] = dis * fo


def _tc_final(ts_ref, x_ref, fo_ref, part_ref, u1_ref, dis_ref,
              w_ref, re_ref, im_ref):
    n, d = x_ref.shape
    ts = ts_ref[0, 0]
    sl = pl.ds(0, n)
    so = dis_ref[sl, :] * (part_ref[0, sl, :] + part_ref[1, sl, :]
                           + u1_ref[sl, :])
    re = x_ref[...] - (0.5 * ts * ts) * so
    im = ts * fo_ref[sl, :]
    w = jnp.sum(re * re + im * im, axis=1, keepdims=True)
    total = jnp.sum(w)
    wn = jnp.where(total > 1e-8, w * (float(n) / total), jnp.ones_like(w))
    w_ref[...] = wn
    re_ref[...] = re
    im_ref[...] = im


def kernel(x_complex, edge_index, evolution_time, diffusion_strength):
    n, d = x_complex.shape
    e = edge_index.shape[1]
    n_pad = ((n + NS * 128 - 1) // (NS * 128)) * (NS * 128)
    per_sub = n_pad // NS

    edge_p = edge_index
    if e % GROUP:
        pad_len = GROUP - e % GROUP
        pad_idx = jnp.full((2, pad_len), n_pad - 1, dtype=jnp.int32)
        edge_p = jnp.concatenate([edge_index, pad_idx], axis=1)
    e_pad = edge_p.shape[1]
    tot_g = e_pad // GROUP
    q, r = divmod(tot_g, NW)

    zeros2d = jnp.zeros((per_sub, D), jnp.float32)
    zeros1d = jnp.zeros((per_sub,), jnp.float32)

    mesh = plsc.VectorSubcoreMesh(core_axis_name="c", subcore_axis_name="s")
    idx_words = (q + (1 if r else 0)) * GROUP

    hist0, hist1 = pl.kernel(
        functools.partial(_sc_hist, q=q, r=r, per_sub=per_sub),
        mesh=mesh,
        out_type=(jax.ShapeDtypeStruct((n_pad,), jnp.float32),
                  jax.ShapeDtypeStruct((n_pad,), jnp.float32)),
        scratch_types=[
            pltpu.VMEM((idx_words,), jnp.int32),
            pltpu.VMEM((GROUP,), jnp.int32),
            pltpu.VMEM((GROUP,), jnp.float32),
            pltpu.VMEM_SHARED((n_pad,), jnp.float32),
            pltpu.SemaphoreType.DMA,
        ],
    )(edge_p, zeros1d)

    deg = hist0 + hist1 + 1.0
    dis = jax.lax.rsqrt(deg).reshape(n_pad, 1)

    sc_pass = pl.kernel(
        functools.partial(_sc_pass, q=q, r=r, per_sub=per_sub),
        mesh=mesh,
        out_type=jax.ShapeDtypeStruct((NC, n_pad, D), jnp.float32),
        scratch_types=[
            pltpu.VMEM((idx_words,), jnp.int32),
            pltpu.VMEM((GROUP,), jnp.int32),
            pltpu.VMEM((GROUP,), jnp.int32),
            pltpu.VMEM((GROUP, D), jnp.float32),
            pltpu.VMEM((GROUP, D), jnp.float32),
            pltpu.SemaphoreType.DMA,
            pltpu.SemaphoreType.DMA,
            pltpu.SemaphoreType.DMA,
            pltpu.SemaphoreType.DMA,
            pltpu.SemaphoreType.DMA,
            pltpu.SemaphoreType.DMA,
            pltpu.VMEM_SHARED((n_pad, D), jnp.float32),
        ],
    )

    u0 = pl.pallas_call(
        _tc_prep,
        out_shape=jax.ShapeDtypeStruct((n_pad, D), jnp.float32),
    )(dis, x_complex)

    part1 = sc_pass(u0, edge_p, zeros2d)

    fo, u1 = pl.pallas_call(
        _tc_mid,
        out_shape=(jax.ShapeDtypeStruct((n_pad, D), jnp.float32),
                   jax.ShapeDtypeStruct((n_pad, D), jnp.float32)),
    )(part1, u0, dis)

    part2 = sc_pass(u1, edge_p, zeros2d)

    ts = (evolution_time * diffusion_strength).astype(jnp.float32).reshape(1, 1)
    w, re, im = pl.pallas_call(
        _tc_final,
        out_shape=(jax.ShapeDtypeStruct((n, 1), jnp.float32),
                   jax.ShapeDtypeStruct((n, d), jnp.float32),
                   jax.ShapeDtypeStruct((n, d), jnp.float32)),
    )(ts, x_complex, fo, part2, u1, dis)
    evolved = jnp.stack([re, im], axis=-1).reshape(n, 2 * d).view(jnp.complex64)
    return w, evolved


# speedup vs baseline: 1.1756x; 1.1756x over previous
import functools

import jax
import jax.numpy as jnp
from jax import lax
from jax.experimental import pallas as pl
from jax.experimental.pallas import tpu as pltpu
from jax.experimental.pallas import tpu_sc as plsc

D = 128
GROUP = 128
NC = 2
NS = 16
NW = NC * NS


def _load_worker_idx(src_hbm, dst_v, bw, q, r, wid):
    pltpu.sync_copy(src_hbm.at[pl.ds(bw * GROUP, q * GROUP)],
                    dst_v.at[pl.ds(0, q * GROUP)])
    if r:
        @pl.when(wid < r)
        def _():
            pltpu.sync_copy(src_hbm.at[pl.ds((bw + q) * GROUP, GROUP)],
                            dst_v.at[pl.ds(q * GROUP, GROUP)])


def _sc_hist(edge_hbm, zeros_hbm, out0_hbm, out1_hbm, cidx_v, cbuf_v, ones_v,
             acc, sem, *, q, r, per_sub):
    c = lax.axis_index("c")
    s = lax.axis_index("s")
    wid = c * NS + s
    bw = wid * q + jnp.minimum(wid, r)
    tw = q + (wid < r).astype(jnp.int32)
    for j in range(GROUP // 16):
        ones_v[pl.ds(j * 16, 16)] = jnp.ones((16,), jnp.float32)
    _load_worker_idx(edge_hbm.at[1], cidx_v, bw, q, r, wid)
    pltpu.sync_copy(zeros_hbm, acc.at[pl.ds(s * per_sub, per_sub)])
    plsc.subcore_barrier()

    def body(g, carry):
        for j in range(GROUP // 16):
            cbuf_v[pl.ds(j * 16, 16)] = cidx_v[pl.ds(g * GROUP + j * 16, 16)]
        pltpu.sync_copy(ones_v, acc.at[cbuf_v], add=True)
        return carry

    lax.fori_loop(0, tw, body, 0)
    plsc.subcore_barrier()

    @pl.when(c == 0)
    def _():
        pltpu.sync_copy(acc.at[pl.ds(s * per_sub, per_sub)],
                        out0_hbm.at[pl.ds(s * per_sub, per_sub)])

    @pl.when(c == 1)
    def _():
        pltpu.sync_copy(acc.at[pl.ds(s * per_sub, per_sub)],
                        out1_hbm.at[pl.ds(s * per_sub, per_sub)])


def _sc_pass(u_hbm, edge_hbm, zeros_hbm, out_hbm,
             cidx_v, ridx0, ridx1, rows0, rows1,
             gsem0, gsem1, rsem0, rsem1, ssem0, ssem1, acc,
             *, q, r, per_sub):
    c = lax.axis_index("c")
    s = lax.axis_index("s")
    wid = c * NS + s
    bw = wid * q + jnp.minimum(wid, r)
    tw = q + (wid < r).astype(jnp.int32)
    _load_worker_idx(edge_hbm.at[1], cidx_v, bw, q, r, wid)
    pltpu.sync_copy(zeros_hbm, acc.at[pl.ds(s * per_sub, per_sub)])
    plsc.subcore_barrier()

    bufs = ((rows0, gsem0, ridx0, rsem0, ssem0),
            (rows1, gsem1, ridx1, rsem1, ssem1))

    def gather_of(g, rows, gsem):
        return pltpu.make_async_copy(
            u_hbm.at[cidx_v.at[pl.ds(g * GROUP, GROUP)]], rows, gsem)

    def ridx_of(g, ridx, rsem):
        return pltpu.make_async_copy(
            edge_hbm.at[0, pl.ds((bw + g) * GROUP, GROUP)], ridx, rsem)

    def scatter_of(rows, ridx, ssem):
        return pltpu.make_async_copy(rows, acc.at[ridx], ssem)

    @pl.when(tw > 0)
    def _():
        ridx_of(0, ridx0, rsem0).start()
        gather_of(0, rows0, gsem0).start()

    def body(g, carry):
        for b in range(2):
            @pl.when(lax.rem(g, 2) == b)
            def _(b=b):
                rows, gsem, ridx, rsem, ssem = bufs[b]
                orows, ogsem, oridx, orsem, ossem = bufs[1 - b]

                gather_of(g, rows, gsem).wait()

                @pl.when(g >= 1)
                def _():
                    scatter_of(orows, oridx, ossem).wait()

                @pl.when(g + 1 < tw)
                def _():
                    ridx_of(g + 1, oridx, orsem).start()
                    gather_of(g + 1, orows, ogsem).start()

                ridx_of(g, ridx, rsem).wait()
                pltpu.async_copy(rows, acc.at[ridx], ssem, add=True)
        return carry

    lax.fori_loop(0, tw, body, 0)
    for b in range(2):
        @pl.when(jnp.logical_and(tw > 0, lax.rem(tw - 1, 2) == b))
        def _(b=b):
            rows, _gs, ridx, _rs, ssem = bufs[b]
            scatter_of(rows, ridx, ssem).wait()
    plsc.subcore_barrier()
    pltpu.sync_copy(acc.at[pl.ds(s * per_sub, per_sub)],
                    out_hbm.at[c, pl.ds(s * per_sub, per_sub)])


def _tc_prep(dis_ref, x_ref, u0_ref):
    n = x_ref.shape[0]
    n_pad = u0_ref.shape[0]
    u0_ref[pl.ds(0, n), :] = dis_ref[pl.ds(0, n), :] * x_ref[...]
    u0_ref[pl.ds(n, n_pad - n), :] = jnp.zeros((n_pad - n, D), jnp.float32)


def _tc_mid(part_ref, u0_ref, dis_ref, fo_ref, u1_ref):
    v1 = part_ref[0] + part_ref[1] + u0_ref[...]
    dis = dis_ref[...]
    fo = dis * v1
    fo_ref[...] = fo
    u1_ref[...] = (dis * fo).astype(u1_ref.dtype)


def _tc_final(ts_ref, x_ref, fo_ref, part_ref, u1_ref, dis_ref,
              w_ref, re_ref, im_ref):
    n, d = x_ref.shape
    ts = ts_ref[0, 0]
    sl = pl.ds(0, n)
    so = dis_ref[sl, :] * (part_ref[0, sl, :].astype(jnp.float32)
                           + part_ref[1, sl, :].astype(jnp.float32)
                           + u1_ref[sl, :].astype(jnp.float32))
    re = x_ref[...] - (0.5 * ts * ts) * so
    im = ts * fo_ref[sl, :]
    w = jnp.sum(re * re + im * im, axis=1, keepdims=True)
    total = jnp.sum(w)
    wn = jnp.where(total > 1e-8, w * (float(n) / total), jnp.ones_like(w))
    w_ref[...] = wn
    re_ref[...] = re
    im_ref[...] = im


def kernel(x_complex, edge_index, evolution_time, diffusion_strength):
    n, d = x_complex.shape
    e = edge_index.shape[1]
    n_pad = ((n + NS * 128 - 1) // (NS * 128)) * (NS * 128)
    per_sub = n_pad // NS

    edge_p = edge_index
    if e % GROUP:
        pad_len = GROUP - e % GROUP
        pad_idx = jnp.full((2, pad_len), n_pad - 1, dtype=jnp.int32)
        edge_p = jnp.concatenate([edge_index, pad_idx], axis=1)
    e_pad = edge_p.shape[1]
    tot_g = e_pad // GROUP
    q, r = divmod(tot_g, NW)

    zeros2d = jnp.zeros((per_sub, D), jnp.float32)
    zeros1d = jnp.zeros((per_sub,), jnp.float32)

    mesh = plsc.VectorSubcoreMesh(core_axis_name="c", subcore_axis_name="s")
    idx_words = (q + (1 if r else 0)) * GROUP

    hist0, hist1 = pl.kernel(
        functools.partial(_sc_hist, q=q, r=r, per_sub=per_sub),
        mesh=mesh,
        out_type=(jax.ShapeDtypeStruct((n_pad,), jnp.float32),
                  jax.ShapeDtypeStruct((n_pad,), jnp.float32)),
        scratch_types=[
            pltpu.VMEM((idx_words,), jnp.int32),
            pltpu.VMEM((GROUP,), jnp.int32),
            pltpu.VMEM((GROUP,), jnp.float32),
            pltpu.VMEM_SHARED((n_pad,), jnp.float32),
            pltpu.SemaphoreType.DMA,
        ],
    )(edge_p, zeros1d)

    deg = hist0 + hist1 + 1.0
    dis = jax.lax.rsqrt(deg).reshape(n_pad, 1)

    def make_pass(dtype):
        return pl.kernel(
            functools.partial(_sc_pass, q=q, r=r, per_sub=per_sub),
            mesh=mesh,
            out_type=jax.ShapeDtypeStruct((NC, n_pad, D), dtype),
            scratch_types=[
                pltpu.VMEM((idx_words,), jnp.int32),
                pltpu.VMEM((GROUP,), jnp.int32),
                pltpu.VMEM((GROUP,), jnp.int32),
                pltpu.VMEM((GROUP, D), dtype),
                pltpu.VMEM((GROUP, D), dtype),
                pltpu.SemaphoreType.DMA,
                pltpu.SemaphoreType.DMA,
                pltpu.SemaphoreType.DMA,
                pltpu.SemaphoreType.DMA,
                pltpu.SemaphoreType.DMA,
                pltpu.SemaphoreType.DMA,
                pltpu.VMEM_SHARED((n_pad, D), dtype),
            ],
        )

    sc_pass = make_pass(jnp.float32)

    u0 = pl.pallas_call(
        _tc_prep,
        out_shape=jax.ShapeDtypeStruct((n_pad, D), jnp.float32),
    )(dis, x_complex)

    part1 = sc_pass(u0, edge_p, zeros2d)

    fo, u1 = pl.pallas_call(
        _tc_mid,
        out_shape=(jax.ShapeDtypeStruct((n_pad, D), jnp.float32),
                   jax.ShapeDtypeStruct((n_pad, D), jnp.float32)),
    )(part1, u0, dis)

    part2 = sc_pass(u1, edge_p, zeros2d)

    ts = (evolution_time * diffusion_strength).astype(jnp.float32).reshape(1, 1)
    w, re, im = pl.pallas_call(
        _tc_final,
        out_shape=(jax.ShapeDtypeStruct((n, 1), jnp.float32),
                   jax.ShapeDtypeStruct((n, d), jnp.float32),
                   jax.ShapeDtypeStruct((n, d), jnp.float32)),
    )(ts, x_complex, fo, part2, u1, dis)
    return w, jax.lax.complex(re, im)
